# baseline (speedup 1.0000x reference)
import functools

import jax
import jax.numpy as jnp
from jax import lax
from jax.experimental import pallas as pl
from jax.experimental.pallas import tpu as pltpu
from jax.experimental.pallas import tpu_sc as plsc

N_ANCHORS = 1024
DIM = 128
K = 10
MARGIN = 0.5
NEG_FILL = -3.0
BLOCK_W = 1000
N_BLOCKS = 100000 // BLOCK_W


def _make_sc_gather():
    info = plsc.get_sparse_core_info()
    nc, ns = info.num_cores, info.num_subcores
    nw = nc * ns
    b_per_w = N_ANCHORS // nw

    mesh = plsc.VectorSubcoreMesh(core_axis_name="c", subcore_axis_name="s")

    @functools.partial(
        pl.kernel,
        mesh=mesh,
        out_type=[
            jax.ShapeDtypeStruct((N_ANCHORS, DIM), jnp.float32),
            jax.ShapeDtypeStruct((N_ANCHORS, DIM), jnp.float32),
        ],
        scratch_types=[
            pltpu.VMEM((b_per_w,), jnp.int32),
            pltpu.VMEM((b_per_w,), jnp.int32),
            pltpu.VMEM((b_per_w, DIM), jnp.float32),
            pltpu.VMEM((b_per_w, DIM), jnp.float32),
            pltpu.SemaphoreType.DMA,
            pltpu.SemaphoreType.DMA,
        ],
    )
    def sc_gather(idx1_hbm, idx2_hbm, t1_hbm, t2_hbm, o1_hbm, o2_hbm,
                  idx1_v, idx2_v, rows1_v, rows2_v, sem1, sem2):
        wid = lax.axis_index("s") * nc + lax.axis_index("c")
        base = wid * b_per_w
        pltpu.sync_copy(idx1_hbm.at[pl.ds(base, b_per_w)], idx1_v)
        pltpu.sync_copy(idx2_hbm.at[pl.ds(base, b_per_w)], idx2_v)
        cp1 = pltpu.async_copy(t1_hbm.at[idx1_v], rows1_v, sem1)
        cp2 = pltpu.async_copy(t2_hbm.at[idx2_v], rows2_v, sem2)
        cp1.wait()
        cp2.wait()
        pltpu.sync_copy(rows1_v, o1_hbm.at[pl.ds(base, b_per_w)])
        pltpu.sync_copy(rows2_v, o2_hbm.at[pl.ds(base, b_per_w)])

    return sc_gather


_sc_gather_cache = []


def _sc_gather(anchor1, anchor2, out1, out2):
    if not _sc_gather_cache:
        _sc_gather_cache.append(_make_sc_gather())
    return _sc_gather_cache[0](anchor1, anchor2, out1, out2)


N_TILES = (BLOCK_W + 127) // 128
RCHUNK = 64


def _tc_side_body(anc_in_ref, oth_ref, tbl_ref, out_ref, anc_ref,
                  a0_ref, b0_ref, a1_ref, b1_ref):
    j = pl.program_id(0)
    last = N_BLOCKS - 1

    def _normalize_bf16(x):
        nrm = jnp.maximum(
            jnp.sqrt(jnp.sum(x * x, axis=1, keepdims=True)), 1e-12)
        return (x / nrm).astype(jnp.bfloat16)

    @pl.when(j == 0)
    def _init():
        anc_ref[...] = _normalize_bf16(anc_in_ref[...])
        fill = jnp.full((N_ANCHORS, 128), NEG_FILL, jnp.float32)
        a0_ref[...] = fill
        b0_ref[...] = fill
        a1_ref[...] = fill
        b1_ref[...] = fill

    blkn = _normalize_bf16(tbl_ref[...])
    sims = lax.dot_general(
        anc_ref[...], blkn, (((1,), (1,)), ((), ())),
        preferred_element_type=jnp.float32)
    pad = jnp.full((N_ANCHORS, 128 - (BLOCK_W - (N_TILES - 1) * 128)),
                   NEG_FILL, jnp.float32)
    pa = [a0_ref[...], a1_ref[...]]
    pb = [b0_ref[...], b1_ref[...]]
    for t in range(N_TILES):
        lo = t * 128
        hi = min(lo + 128, BLOCK_W)
        tile = sims[:, lo:hi]
        if hi - lo < 128:
            tile = jnp.concatenate([tile, pad], axis=1)
        p = t & 1
        rr = jnp.minimum(pa[p], tile)
        pa[p] = jnp.maximum(pa[p], tile)
        pb[p] = jnp.maximum(pb[p], rr)
    a0_ref[...] = pa[0]
    a1_ref[...] = pa[1]
    b0_ref[...] = pb[0]
    b1_ref[...] = pb[1]

    @pl.when(j == last)
    def _side_loss():
        x1 = anc_in_ref[...]
        x2 = oth_ref[...]
        num = jnp.sum(x1 * x2, axis=1, keepdims=True)
        den = (jnp.sqrt(jnp.sum(x1 * x1, axis=1, keepdims=True)) *
               jnp.sqrt(jnp.sum(x2 * x2, axis=1, keepdims=True)))
        d_m1 = (1.0 + MARGIN - num / den) - 1.0
        tot = jnp.float32(0.0)
        for r in range(N_ANCHORS // RCHUNK):
            rows = pl.ds(r * RCHUNK, RCHUNK)
            cands = [a0_ref[rows, :], a1_ref[rows, :],
                     b0_ref[rows, :], b1_ref[rows, :]]
            dch = d_m1[r * RCHUNK:(r + 1) * RCHUNK, :]
            ch = jnp.zeros((RCHUNK, 1), jnp.float32)
            for _ in range(K):
                m = None
                for cd in cands:
                    mm = jnp.max(cd, axis=1, keepdims=True)
                    m = mm if m is None else jnp.maximum(m, mm)
                ch += jnp.maximum(dch + m, 0.0)
                cands = [jnp.where(cd == m, NEG_FILL, cd) for cd in cands]
            tot += jnp.sum(ch)
        out_ref[...] = jnp.broadcast_to(tot / (N_ANCHORS * K), (1, 1))


def _tc_side_loss(anchors, others, table):
    return pl.pallas_call(
        _tc_side_body,
        grid=(N_BLOCKS,),
        in_specs=[
            pl.BlockSpec((N_ANCHORS, DIM), lambda j: (0, 0)),
            pl.BlockSpec((N_ANCHORS, DIM), lambda j: (0, 0)),
            pl.BlockSpec((BLOCK_W, DIM), lambda j: (j, 0)),
        ],
        out_specs=pl.BlockSpec((1, 1), lambda j: (0, 0)),
        out_shape=jax.ShapeDtypeStruct((1, 1), jnp.float32),
        scratch_shapes=[
            pltpu.VMEM((N_ANCHORS, DIM), jnp.bfloat16),
            pltpu.VMEM((N_ANCHORS, 128), jnp.float32),
            pltpu.VMEM((N_ANCHORS, 128), jnp.float32),
            pltpu.VMEM((N_ANCHORS, 128), jnp.float32),
            pltpu.VMEM((N_ANCHORS, 128), jnp.float32),
        ],
    )(anchors, others, table)


def kernel(out1, out2, anchor_links):
    anchor1 = anchor_links[:, 0].astype(jnp.int32)
    anchor2 = anchor_links[:, 1].astype(jnp.int32)
    a1, a2 = _sc_gather(anchor1, anchor2, out1, out2)
    p1 = _tc_side_loss(a1, a2, out2)
    p2 = _tc_side_loss(a2, a1, out1)
    return p1[0, 0] + p2[0, 0]


# speedup vs baseline: 1.0784x; 1.0784x over previous
import functools

import jax
import jax.numpy as jnp
from jax import lax
from jax.experimental import pallas as pl
from jax.experimental.pallas import tpu as pltpu
from jax.experimental.pallas import tpu_sc as plsc

N_ANCHORS = 1024
DIM = 128
K = 10
MARGIN = 0.5
NEG_FILL = -3.0
BLOCK_W = 1000
N_BLOCKS = 100000 // BLOCK_W


def _make_sc_gather():
    info = plsc.get_sparse_core_info()
    nc, ns = info.num_cores, info.num_subcores
    nw = nc * ns
    b_per_w = N_ANCHORS // nw

    mesh = plsc.VectorSubcoreMesh(core_axis_name="c", subcore_axis_name="s")

    @functools.partial(
        pl.kernel,
        mesh=mesh,
        out_type=[
            jax.ShapeDtypeStruct((N_ANCHORS, DIM), jnp.float32),
            jax.ShapeDtypeStruct((N_ANCHORS, DIM), jnp.float32),
        ],
        scratch_types=[
            pltpu.VMEM((b_per_w,), jnp.int32),
            pltpu.VMEM((b_per_w,), jnp.int32),
            pltpu.VMEM((b_per_w, DIM), jnp.float32),
            pltpu.VMEM((b_per_w, DIM), jnp.float32),
            pltpu.SemaphoreType.DMA,
            pltpu.SemaphoreType.DMA,
        ],
    )
    def sc_gather(idx1_hbm, idx2_hbm, t1_hbm, t2_hbm, o1_hbm, o2_hbm,
                  idx1_v, idx2_v, rows1_v, rows2_v, sem1, sem2):
        wid = lax.axis_index("s") * nc + lax.axis_index("c")
        base = wid * b_per_w
        pltpu.sync_copy(idx1_hbm.at[pl.ds(base, b_per_w)], idx1_v)
        pltpu.sync_copy(idx2_hbm.at[pl.ds(base, b_per_w)], idx2_v)
        cp1 = pltpu.async_copy(t1_hbm.at[idx1_v], rows1_v, sem1)
        cp2 = pltpu.async_copy(t2_hbm.at[idx2_v], rows2_v, sem2)
        cp1.wait()
        cp2.wait()
        pltpu.sync_copy(rows1_v, o1_hbm.at[pl.ds(base, b_per_w)])
        pltpu.sync_copy(rows2_v, o2_hbm.at[pl.ds(base, b_per_w)])

    return sc_gather


_sc_gather_cache = []


def _sc_gather(anchor1, anchor2, out1, out2):
    if not _sc_gather_cache:
        _sc_gather_cache.append(_make_sc_gather())
    return _sc_gather_cache[0](anchor1, anchor2, out1, out2)


N_TILES = (BLOCK_W + 127) // 128
RCHUNK = 64


def _tc_side_body(anc_in_ref, oth_ref, tbl_ref, out_ref, anc_ref,
                  a0_ref, b0_ref, a1_ref, b1_ref):
    j = pl.program_id(0)
    last = N_BLOCKS - 1

    def _normalize_bf16(x):
        nrm = jnp.maximum(
            jnp.sqrt(jnp.sum(x * x, axis=1, keepdims=True)), 1e-12)
        return (x / nrm).astype(jnp.bfloat16)

    @pl.when(j == 0)
    def _init():
        anc_ref[...] = _normalize_bf16(anc_in_ref[...])
        fill = jnp.full((N_ANCHORS, 128), NEG_FILL, jnp.float32)
        a0_ref[...] = fill
        b0_ref[...] = fill
        a1_ref[...] = fill
        b1_ref[...] = fill

    blkn = _normalize_bf16(tbl_ref[...])
    sims = lax.dot_general(
        anc_ref[...], blkn, (((1,), (1,)), ((), ())),
        preferred_element_type=jnp.float32)
    pad = jnp.full((N_ANCHORS, 128 - (BLOCK_W - (N_TILES - 1) * 128)),
                   NEG_FILL, jnp.float32)
    pa = [a0_ref[...], a1_ref[...]]
    pb = [b0_ref[...], b1_ref[...]]
    for t in range(N_TILES):
        lo = t * 128
        hi = min(lo + 128, BLOCK_W)
        tile = sims[:, lo:hi]
        if hi - lo < 128:
            tile = jnp.concatenate([tile, pad], axis=1)
        p = t & 1
        rr = jnp.minimum(pa[p], tile)
        pa[p] = jnp.maximum(pa[p], tile)
        pb[p] = jnp.maximum(pb[p], rr)
    a0_ref[...] = pa[0]
    a1_ref[...] = pa[1]
    b0_ref[...] = pb[0]
    b1_ref[...] = pb[1]

    @pl.when(j == last)
    def _side_loss():
        x1 = anc_in_ref[...]
        x2 = oth_ref[...]
        num = jnp.sum(x1 * x2, axis=1, keepdims=True)
        den = (jnp.sqrt(jnp.sum(x1 * x1, axis=1, keepdims=True)) *
               jnp.sqrt(jnp.sum(x2 * x2, axis=1, keepdims=True)))
        d_m1 = (1.0 + MARGIN - num / den) - 1.0
        cands = [a0_ref[...], a1_ref[...], b0_ref[...], b1_ref[...]]
        tot = jnp.zeros((N_ANCHORS, 1), jnp.float32)
        for _ in range(K):
            m = None
            for cd in cands:
                mm = jnp.max(cd, axis=1, keepdims=True)
                m = mm if m is None else jnp.maximum(m, mm)
            tot += jnp.maximum(d_m1 + m, 0.0)
            cands = [jnp.where(cd == m, NEG_FILL, cd) for cd in cands]
        out_ref[...] = jnp.broadcast_to(
            jnp.sum(tot) / (N_ANCHORS * K), (1, 1))


def _tc_side_loss(anchors, others, table):
    return pl.pallas_call(
        _tc_side_body,
        grid=(N_BLOCKS,),
        in_specs=[
            pl.BlockSpec((N_ANCHORS, DIM), lambda j: (0, 0)),
            pl.BlockSpec((N_ANCHORS, DIM), lambda j: (0, 0)),
            pl.BlockSpec((BLOCK_W, DIM), lambda j: (j, 0)),
        ],
        out_specs=pl.BlockSpec((1, 1), lambda j: (0, 0)),
        out_shape=jax.ShapeDtypeStruct((1, 1), jnp.float32),
        scratch_shapes=[
            pltpu.VMEM((N_ANCHORS, DIM), jnp.bfloat16),
            pltpu.VMEM((N_ANCHORS, 128), jnp.float32),
            pltpu.VMEM((N_ANCHORS, 128), jnp.float32),
            pltpu.VMEM((N_ANCHORS, 128), jnp.float32),
            pltpu.VMEM((N_ANCHORS, 128), jnp.float32),
        ],
    )(anchors, others, table)


def kernel(out1, out2, anchor_links):
    anchor1 = anchor_links[:, 0].astype(jnp.int32)
    anchor2 = anchor_links[:, 1].astype(jnp.int32)
    a1, a2 = _sc_gather(anchor1, anchor2, out1, out2)
    p1 = _tc_side_loss(a1, a2, out2)
    p2 = _tc_side_loss(a2, a1, out1)
    return p1[0, 0] + p2[0, 0]


# speedup vs baseline: 1.2624x; 1.1706x over previous
import functools

import jax
import jax.numpy as jnp
from jax import lax
from jax.experimental import pallas as pl
from jax.experimental.pallas import tpu as pltpu
from jax.experimental.pallas import tpu_sc as plsc

N_ANCHORS = 1024
DIM = 128
K = 10
MARGIN = 0.5
NEG_FILL = -3.0
BLOCK_W = 2000
N_BLOCKS = 100000 // BLOCK_W


def _make_sc_gather():
    info = plsc.get_sparse_core_info()
    nc, ns = info.num_cores, info.num_subcores
    nw = nc * ns
    b_per_w = N_ANCHORS // nw

    mesh = plsc.VectorSubcoreMesh(core_axis_name="c", subcore_axis_name="s")

    @functools.partial(
        pl.kernel,
        mesh=mesh,
        out_type=[
            jax.ShapeDtypeStruct((N_ANCHORS, DIM), jnp.float32),
            jax.ShapeDtypeStruct((N_ANCHORS, DIM), jnp.float32),
        ],
        scratch_types=[
            pltpu.VMEM((b_per_w,), jnp.int32),
            pltpu.VMEM((b_per_w,), jnp.int32),
            pltpu.VMEM((b_per_w, DIM), jnp.float32),
            pltpu.VMEM((b_per_w, DIM), jnp.float32),
            pltpu.SemaphoreType.DMA,
            pltpu.SemaphoreType.DMA,
        ],
    )
    def sc_gather(idx1_hbm, idx2_hbm, t1_hbm, t2_hbm, o1_hbm, o2_hbm,
                  idx1_v, idx2_v, rows1_v, rows2_v, sem1, sem2):
        wid = lax.axis_index("s") * nc + lax.axis_index("c")
        base = wid * b_per_w
        pltpu.sync_copy(idx1_hbm.at[pl.ds(base, b_per_w)], idx1_v)
        pltpu.sync_copy(idx2_hbm.at[pl.ds(base, b_per_w)], idx2_v)
        cp1 = pltpu.async_copy(t1_hbm.at[idx1_v], rows1_v, sem1)
        cp2 = pltpu.async_copy(t2_hbm.at[idx2_v], rows2_v, sem2)
        cp1.wait()
        cp2.wait()
        pltpu.sync_copy(rows1_v, o1_hbm.at[pl.ds(base, b_per_w)])
        pltpu.sync_copy(rows2_v, o2_hbm.at[pl.ds(base, b_per_w)])

    return sc_gather


_sc_gather_cache = []


def _sc_gather(anchor1, anchor2, out1, out2):
    if not _sc_gather_cache:
        _sc_gather_cache.append(_make_sc_gather())
    return _sc_gather_cache[0](anchor1, anchor2, out1, out2)


N_TILES = (BLOCK_W + 127) // 128
RCHUNK = 64


def _tc_side_body(anc_in_ref, oth_ref, tbl_ref, out_ref, anc_ref,
                  a0_ref, b0_ref, a1_ref, b1_ref):
    j = pl.program_id(0)
    last = N_BLOCKS - 1

    def _normalize_bf16(x):
        nrm = jnp.maximum(
            jnp.sqrt(jnp.sum(x * x, axis=1, keepdims=True)), 1e-12)
        return (x / nrm).astype(jnp.bfloat16)

    @pl.when(j == 0)
    def _init():
        anc_ref[...] = _normalize_bf16(anc_in_ref[...])
        fill = jnp.full((N_ANCHORS, 128), NEG_FILL, jnp.float32)
        a0_ref[...] = fill
        b0_ref[...] = fill
        a1_ref[...] = fill
        b1_ref[...] = fill

    blkn = _normalize_bf16(tbl_ref[...])
    sims = lax.dot_general(
        anc_ref[...], blkn, (((1,), (1,)), ((), ())),
        preferred_element_type=jnp.float32)
    pad = jnp.full((N_ANCHORS, 128 - (BLOCK_W - (N_TILES - 1) * 128)),
                   NEG_FILL, jnp.float32)
    pa = [a0_ref[...], a1_ref[...]]
    pb = [b0_ref[...], b1_ref[...]]
    for t in range(N_TILES):
        lo = t * 128
        hi = min(lo + 128, BLOCK_W)
        tile = sims[:, lo:hi]
        if hi - lo < 128:
            tile = jnp.concatenate([tile, pad], axis=1)
        p = t & 1
        rr = jnp.minimum(pa[p], tile)
        pa[p] = jnp.maximum(pa[p], tile)
        pb[p] = jnp.maximum(pb[p], rr)
    a0_ref[...] = pa[0]
    a1_ref[...] = pa[1]
    b0_ref[...] = pb[0]
    b1_ref[...] = pb[1]

    @pl.when(j == last)
    def _side_loss():
        x1 = anc_in_ref[...]
        x2 = oth_ref[...]
        num = jnp.sum(x1 * x2, axis=1, keepdims=True)
        den = (jnp.sqrt(jnp.sum(x1 * x1, axis=1, keepdims=True)) *
               jnp.sqrt(jnp.sum(x2 * x2, axis=1, keepdims=True)))
        d_m1 = (1.0 + MARGIN - num / den) - 1.0
        cands = [a0_ref[...], a1_ref[...], b0_ref[...], b1_ref[...]]
        tot = jnp.zeros((N_ANCHORS, 1), jnp.float32)
        for _ in range(K):
            m = None
            for cd in cands:
                mm = jnp.max(cd, axis=1, keepdims=True)
                m = mm if m is None else jnp.maximum(m, mm)
            tot += jnp.maximum(d_m1 + m, 0.0)
            cands = [jnp.where(cd == m, NEG_FILL, cd) for cd in cands]
        out_ref[...] = jnp.broadcast_to(
            jnp.sum(tot) / (N_ANCHORS * K), (1, 1))


def _tc_side_loss(anchors, others, table):
    return pl.pallas_call(
        _tc_side_body,
        grid=(N_BLOCKS,),
        in_specs=[
            pl.BlockSpec((N_ANCHORS, DIM), lambda j: (0, 0)),
            pl.BlockSpec((N_ANCHORS, DIM), lambda j: (0, 0)),
            pl.BlockSpec((BLOCK_W, DIM), lambda j: (j, 0)),
        ],
        out_specs=pl.BlockSpec((1, 1), lambda j: (0, 0)),
        out_shape=jax.ShapeDtypeStruct((1, 1), jnp.float32),
        scratch_shapes=[
            pltpu.VMEM((N_ANCHORS, DIM), jnp.bfloat16),
            pltpu.VMEM((N_ANCHORS, 128), jnp.float32),
            pltpu.VMEM((N_ANCHORS, 128), jnp.float32),
            pltpu.VMEM((N_ANCHORS, 128), jnp.float32),
            pltpu.VMEM((N_ANCHORS, 128), jnp.float32),
        ],
    )(anchors, others, table)


def kernel(out1, out2, anchor_links):
    anchor1 = anchor_links[:, 0].astype(jnp.int32)
    anchor2 = anchor_links[:, 1].astype(jnp.int32)
    a1, a2 = _sc_gather(anchor1, anchor2, out1, out2)
    p1 = _tc_side_loss(a1, a2, out2)
    p2 = _tc_side_loss(a2, a1, out1)
    return p1[0, 0] + p2[0, 0]


# speedup vs baseline: 1.3392x; 1.0608x over previous
import functools

import jax
import jax.numpy as jnp
from jax import lax
from jax.experimental import pallas as pl
from jax.experimental.pallas import tpu as pltpu
from jax.experimental.pallas import tpu_sc as plsc

N_ANCHORS = 1024
DIM = 128
K = 10
MARGIN = 0.5
NEG_FILL = -3.0
BLOCK_W = 4000
N_BLOCKS = 100000 // BLOCK_W


def _make_sc_gather():
    info = plsc.get_sparse_core_info()
    nc, ns = info.num_cores, info.num_subcores
    nw = nc * ns
    b_per_w = N_ANCHORS // nw

    mesh = plsc.VectorSubcoreMesh(core_axis_name="c", subcore_axis_name="s")

    @functools.partial(
        pl.kernel,
        mesh=mesh,
        out_type=[
            jax.ShapeDtypeStruct((N_ANCHORS, DIM), jnp.float32),
            jax.ShapeDtypeStruct((N_ANCHORS, DIM), jnp.float32),
        ],
        scratch_types=[
            pltpu.VMEM((b_per_w,), jnp.int32),
            pltpu.VMEM((b_per_w,), jnp.int32),
            pltpu.VMEM((b_per_w, DIM), jnp.float32),
            pltpu.VMEM((b_per_w, DIM), jnp.float32),
            pltpu.SemaphoreType.DMA,
            pltpu.SemaphoreType.DMA,
        ],
    )
    def sc_gather(idx1_hbm, idx2_hbm, t1_hbm, t2_hbm, o1_hbm, o2_hbm,
                  idx1_v, idx2_v, rows1_v, rows2_v, sem1, sem2):
        wid = lax.axis_index("s") * nc + lax.axis_index("c")
        base = wid * b_per_w
        pltpu.sync_copy(idx1_hbm.at[pl.ds(base, b_per_w)], idx1_v)
        pltpu.sync_copy(idx2_hbm.at[pl.ds(base, b_per_w)], idx2_v)
        cp1 = pltpu.async_copy(t1_hbm.at[idx1_v], rows1_v, sem1)
        cp2 = pltpu.async_copy(t2_hbm.at[idx2_v], rows2_v, sem2)
        cp1.wait()
        cp2.wait()
        pltpu.sync_copy(rows1_v, o1_hbm.at[pl.ds(base, b_per_w)])
        pltpu.sync_copy(rows2_v, o2_hbm.at[pl.ds(base, b_per_w)])

    return sc_gather


_sc_gather_cache = []


def _sc_gather(anchor1, anchor2, out1, out2):
    if not _sc_gather_cache:
        _sc_gather_cache.append(_make_sc_gather())
    return _sc_gather_cache[0](anchor1, anchor2, out1, out2)


N_TILES = (BLOCK_W + 127) // 128
RCHUNK = 64


def _tc_side_body(anc_in_ref, oth_ref, tbl_ref, out_ref, anc_ref,
                  a0_ref, b0_ref, a1_ref, b1_ref):
    j = pl.program_id(0)
    last = N_BLOCKS - 1

    def _normalize_bf16(x):
        nrm = jnp.maximum(
            jnp.sqrt(jnp.sum(x * x, axis=1, keepdims=True)), 1e-12)
        return (x / nrm).astype(jnp.bfloat16)

    @pl.when(j == 0)
    def _init():
        anc_ref[...] = _normalize_bf16(anc_in_ref[...])
        fill = jnp.full((N_ANCHORS, 128), NEG_FILL, jnp.float32)
        a0_ref[...] = fill
        b0_ref[...] = fill
        a1_ref[...] = fill
        b1_ref[...] = fill

    blkn = _normalize_bf16(tbl_ref[...])
    sims = lax.dot_general(
        anc_ref[...], blkn, (((1,), (1,)), ((), ())),
        preferred_element_type=jnp.float32)
    pad = jnp.full((N_ANCHORS, 128 - (BLOCK_W - (N_TILES - 1) * 128)),
                   NEG_FILL, jnp.float32)
    pa = [a0_ref[...], a1_ref[...]]
    pb = [b0_ref[...], b1_ref[...]]
    for t in range(N_TILES):
        lo = t * 128
        hi = min(lo + 128, BLOCK_W)
        tile = sims[:, lo:hi]
        if hi - lo < 128:
            tile = jnp.concatenate([tile, pad], axis=1)
        p = t & 1
        rr = jnp.minimum(pa[p], tile)
        pa[p] = jnp.maximum(pa[p], tile)
        pb[p] = jnp.maximum(pb[p], rr)
    a0_ref[...] = pa[0]
    a1_ref[...] = pa[1]
    b0_ref[...] = pb[0]
    b1_ref[...] = pb[1]

    @pl.when(j == last)
    def _side_loss():
        x1 = anc_in_ref[...]
        x2 = oth_ref[...]
        num = jnp.sum(x1 * x2, axis=1, keepdims=True)
        den = (jnp.sqrt(jnp.sum(x1 * x1, axis=1, keepdims=True)) *
               jnp.sqrt(jnp.sum(x2 * x2, axis=1, keepdims=True)))
        d_m1 = (1.0 + MARGIN - num / den) - 1.0
        cands = [a0_ref[...], a1_ref[...], b0_ref[...], b1_ref[...]]
        tot = jnp.zeros((N_ANCHORS, 1), jnp.float32)
        for _ in range(K):
            m = None
            for cd in cands:
                mm = jnp.max(cd, axis=1, keepdims=True)
                m = mm if m is None else jnp.maximum(m, mm)
            tot += jnp.maximum(d_m1 + m, 0.0)
            cands = [jnp.where(cd == m, NEG_FILL, cd) for cd in cands]
        out_ref[...] = jnp.broadcast_to(
            jnp.sum(tot) / (N_ANCHORS * K), (1, 1))


def _tc_side_loss(anchors, others, table):
    return pl.pallas_call(
        _tc_side_body,
        grid=(N_BLOCKS,),
        in_specs=[
            pl.BlockSpec((N_ANCHORS, DIM), lambda j: (0, 0)),
            pl.BlockSpec((N_ANCHORS, DIM), lambda j: (0, 0)),
            pl.BlockSpec((BLOCK_W, DIM), lambda j: (j, 0)),
        ],
        out_specs=pl.BlockSpec((1, 1), lambda j: (0, 0)),
        out_shape=jax.ShapeDtypeStruct((1, 1), jnp.float32),
        scratch_shapes=[
            pltpu.VMEM((N_ANCHORS, DIM), jnp.bfloat16),
            pltpu.VMEM((N_ANCHORS, 128), jnp.float32),
            pltpu.VMEM((N_ANCHORS, 128), jnp.float32),
            pltpu.VMEM((N_ANCHORS, 128), jnp.float32),
            pltpu.VMEM((N_ANCHORS, 128), jnp.float32),
        ],
    )(anchors, others, table)


def kernel(out1, out2, anchor_links):
    anchor1 = anchor_links[:, 0].astype(jnp.int32)
    anchor2 = anchor_links[:, 1].astype(jnp.int32)
    a1, a2 = _sc_gather(anchor1, anchor2, out1, out2)
    p1 = _tc_side_loss(a1, a2, out2)
    p2 = _tc_side_loss(a2, a1, out1)
    return p1[0, 0] + p2[0, 0]
